# Initial kernel scaffold; baseline (speedup 1.0000x reference)
import jax
import jax.numpy as jnp
from jax.experimental import pallas as pl


def kernel(user_id, history, top_genres, item_id, tmdb_genres, history_ts_diff, user_avg_rating, activity, release_year, item_avg_rating, revenue, item_emb, genre_emb, user_emb, u_avg_rating_bucket, activity_bucket, release_year_bucket, i_avg_rating_bucket, revenue_bucket, Wu1, bu1, gu_ln, bu_ln, Wu2, bu2, Wi1, bi1, gi_ln, bi_ln, Wi2, bi2):
    raise NotImplementedError("write your pallas kernel here")


# speedup vs baseline: 9.6970x; 9.6970x over previous
import dataclasses
import functools

import jax
import jax.numpy as jnp
from jax import lax
from jax.experimental import pallas as pl
from jax.experimental.pallas import tpu as pltpu
from jax.experimental.pallas import tpu_sc as plsc

_B, _H, _G, _D, _NB = 16384, 50, 6, 64, 20
_NC, _NS = 2, 16
_NW = _NC * _NS
_IPW = _B // _NW
_CH = 8
_NCHUNK = _IPW // _CH
_RPC = _CH * _H
_L = 16

_BBLK = 1024


def _sc_gather_pool(item_emb, user_emb, hist_flat, ts_flat, uid, iid):
    mesh = plsc.VectorSubcoreMesh(core_axis_name="c", subcore_axis_name="s")
    out_type = (
        jax.ShapeDtypeStruct((_B, _D), jnp.float32),
        jax.ShapeDtypeStruct((_B, _D), jnp.float32),
        jax.ShapeDtypeStruct((_B, _D), jnp.float32),
    )

    cp = pltpu.CompilerParams(use_tc_tiling_on_sc=False)
    if "needs_layout_passes" in pltpu.CompilerParams.__dataclass_fields__:
        cp = dataclasses.replace(cp, needs_layout_passes=False)

    @functools.partial(
        pl.kernel,
        mesh=mesh,
        out_type=out_type,
        compiler_params=cp,
        scratch_types=[
            pltpu.VMEM((_RPC,), jnp.int32),
            pltpu.VMEM((_RPC,), jnp.float32),
            pltpu.VMEM((_RPC,), jnp.float32),
            pltpu.VMEM((_RPC, _D), jnp.float32),
            pltpu.VMEM((_CH, _D), jnp.float32),
            pltpu.VMEM((256,), jnp.int32),
            pltpu.VMEM((256, _D), jnp.float32),
            pltpu.SemaphoreType.DMA,
        ],
    )
    def k(item_hbm, user_hbm, hist_hbm, ts_hbm, uid_hbm, iid_hbm,
          hs_out, u_out, ie_out,
          idx_v, ts_v, w_v, rows_v, sum_v, id_v, row256_v, sem):
        wid = lax.axis_index("s") * _NC + lax.axis_index("c")
        item_base = wid * _IPW

        @pl.loop(0, _NCHUNK)
        def _chunk(ck):
            base_i = item_base + ck * _CH
            base_r = base_i * _H
            pltpu.sync_copy(hist_hbm.at[pl.ds(base_r, _RPC)], idx_v)
            pltpu.sync_copy(ts_hbm.at[pl.ds(base_r, _RPC)], ts_v)
            cp = pltpu.async_copy(item_hbm.at[idx_v], rows_v, sem)

            @pl.loop(0, _RPC // _L)
            def _w(j):
                sl = pl.ds(j * _L, _L)
                w = jnp.exp(ts_v[sl] * jnp.float32(-0.001))
                w_v[sl] = jnp.where(idx_v[sl] > 0, w, jnp.float32(0.0))

            cp.wait()
            for i in range(_CH):
                def hbody(h, accs, i=i):
                    r = i * _H + h
                    wv = plsc.load_gather(w_v, [jnp.full((_L,), r, jnp.int32)])
                    return tuple(
                        accs[c] + wv * rows_v[r, pl.ds(c * _L, _L)]
                        for c in range(_D // _L))

                z = jnp.zeros((_L,), jnp.float32)
                accs = lax.fori_loop(0, _H, hbody, (z,) * (_D // _L))
                for c in range(_D // _L):
                    sum_v[i, pl.ds(c * _L, _L)] = accs[c]
            pltpu.sync_copy(sum_v, hs_out.at[pl.ds(base_i, _CH)])

        @pl.loop(0, 2)
        def _ug(ck):
            b = item_base + ck * 256
            pltpu.sync_copy(uid_hbm.at[pl.ds(b, 256)], id_v)
            pltpu.async_copy(user_hbm.at[id_v], row256_v, sem).wait()
            pltpu.sync_copy(row256_v, u_out.at[pl.ds(b, 256)])

        @pl.loop(0, 2)
        def _ig(ck):
            b = item_base + ck * 256
            pltpu.sync_copy(iid_hbm.at[pl.ds(b, 256)], id_v)
            pltpu.async_copy(item_hbm.at[id_v], row256_v, sem).wait()
            pltpu.sync_copy(row256_v, ie_out.at[pl.ds(b, 256)])

    return k(item_emb, user_emb, hist_flat, ts_flat, uid, iid)


def _towers_body(u_ref, hs_ref, ie_ref, hist_ref, ts_ref, tg_ref, ig_ref,
                 uar_ref, act_ref, ry_ref, iar_ref, rev_ref, bounds_ref,
                 gpad_ref, ub_ref, ab_ref, ryb_ref, iab_ref, rvb_ref,
                 Wu1_ref, bu1_ref, guln_ref, buln_ref, Wu2_ref, bu2_ref,
                 Wi1_ref, bi1_ref, giln_ref, biln_ref, Wi2_ref, bi2_ref,
                 uout_ref, iout_ref):
    f32 = jnp.float32
    lane = lax.broadcasted_iota(jnp.int32, (1, 128), 1)

    def genre_agg(g):
        counts = jnp.zeros((g.shape[0], 128), f32)
        for k in range(_G):
            gk = g[:, k:k + 1]
            counts += ((gk == lane) & (gk > 0)).astype(f32)
        cnt = counts.sum(axis=1, keepdims=True)
        ge = jnp.dot(counts, gpad_ref[...], preferred_element_type=f32)
        return ge / (cnt + 1e-8)

    def bucket(v, tbl_ref):
        idx = jnp.sum((bounds_ref[...] < v).astype(jnp.int32), axis=1,
                      keepdims=True)
        oh = (idx == lane).astype(f32)
        return jnp.dot(oh, tbl_ref[...], preferred_element_type=f32)

    def mlp(c, W1_ref, b1_ref, g_ref, b_ref, W2_ref, b2_ref):
        h = jnp.dot(c, W1_ref[...], preferred_element_type=f32) + b1_ref[...]
        m = h.mean(axis=-1, keepdims=True)
        v = ((h - m) ** 2).mean(axis=-1, keepdims=True)
        h = (h - m) / jnp.sqrt(v + 1e-5) * g_ref[...] + b_ref[...]
        h = jnp.maximum(h, 0.0)
        o = jnp.dot(h, W2_ref[...], preferred_element_type=f32) + b2_ref[...]
        n = jnp.sqrt((o * o).sum(axis=-1, keepdims=True))
        return o / jnp.maximum(n, 1e-12)

    w = jnp.exp(ts_ref[...] * f32(-0.001)) * (hist_ref[...] > 0).astype(f32)
    wsum = w.sum(axis=1, keepdims=True)
    hist_agg = hs_ref[...] / (wsum + 1e-8)
    g_agg = genre_agg(tg_ref[...])
    cont_u = bucket(uar_ref[...], ub_ref) + bucket(act_ref[...], ab_ref)
    cu = jnp.concatenate([u_ref[...], hist_agg, g_agg, cont_u], axis=1)
    uout_ref[...] = mlp(cu, Wu1_ref, bu1_ref, guln_ref, buln_ref,
                        Wu2_ref, bu2_ref)

    g2 = genre_agg(ig_ref[...])
    cont_i = (bucket(ry_ref[...], ryb_ref) + bucket(iar_ref[...], iab_ref)
              + bucket(rev_ref[...], rvb_ref))
    ci = jnp.concatenate([ie_ref[...], g2, cont_i], axis=1)
    iout_ref[...] = mlp(ci, Wi1_ref, bi1_ref, giln_ref, biln_ref,
                        Wi2_ref, bi2_ref)


def _towers(u, hs, ie, hist, ts, tg, ig, uar, act, ry, iar, rev, bounds,
            gpad, ub, ab, ryb, iab, rvb,
            Wu1, bu1, guln, buln, Wu2, bu2,
            Wi1, bi1, giln, biln, Wi2, bi2):
    nblk = _B // _BBLK

    def blk(r, c):
        return pl.BlockSpec((_BBLK, c), lambda i: (i, 0))

    def full(a):
        return pl.BlockSpec(a.shape, lambda i: tuple(0 for _ in a.shape))

    in_specs = [
        blk(_B, _D), blk(_B, _D), blk(_B, _D),
        blk(_B, _H), blk(_B, _H), blk(_B, _G), blk(_B, _G),
        blk(_B, 1), blk(_B, 1), blk(_B, 1), blk(_B, 1), blk(_B, 1),
        full(bounds), full(gpad), full(ub), full(ab), full(ryb), full(iab),
        full(rvb),
        full(Wu1), full(bu1), full(guln), full(buln), full(Wu2), full(bu2),
        full(Wi1), full(bi1), full(giln), full(biln), full(Wi2), full(bi2),
    ]
    out_specs = (blk(_B, _D), blk(_B, _D))
    out_shape = (jax.ShapeDtypeStruct((_B, _D), jnp.float32),
                 jax.ShapeDtypeStruct((_B, _D), jnp.float32))
    return pl.pallas_call(
        _towers_body,
        grid=(nblk,),
        in_specs=in_specs,
        out_specs=out_specs,
        out_shape=out_shape,
    )(u, hs, ie, hist, ts, tg, ig, uar, act, ry, iar, rev, bounds,
      gpad, ub, ab, ryb, iab, rvb,
      Wu1, bu1, guln, buln, Wu2, bu2, Wi1, bi1, giln, biln, Wi2, bi2)


def kernel(user_id, history, top_genres, item_id, tmdb_genres,
           history_ts_diff, user_avg_rating, activity, release_year,
           item_avg_rating, revenue, item_emb, genre_emb, user_emb,
           u_avg_rating_bucket, activity_bucket, release_year_bucket,
           i_avg_rating_bucket, revenue_bucket,
           Wu1, bu1, gu_ln, bu_ln, Wu2, bu2,
           Wi1, bi1, gi_ln, bi_ln, Wi2, bi2):
    f32 = jnp.float32
    hist = history.astype(jnp.int32)
    tg = top_genres.astype(jnp.int32)
    ig = tmdb_genres.astype(jnp.int32)
    uid = user_id.astype(jnp.int32)
    iid = item_id.astype(jnp.int32)
    ts = history_ts_diff.astype(f32)

    hs, u, ie = _sc_gather_pool(item_emb, user_emb, hist.reshape(-1),
                                ts.reshape(-1), uid, iid)

    bounds = jnp.linspace(0.0, 1.0, _NB + 1)[1:-1].reshape(1, _NB - 1)
    bounds = bounds.astype(f32)

    def pad128(t):
        return jnp.pad(t, ((0, 128 - t.shape[0]), (0, 0)))

    row = lambda b: b.reshape(1, -1)
    col = lambda v: v.reshape(-1, 1).astype(f32)

    return _towers(
        u, hs, ie, hist, ts, tg, ig,
        col(user_avg_rating), col(activity), col(release_year),
        col(item_avg_rating), col(revenue), bounds,
        pad128(genre_emb), pad128(u_avg_rating_bucket),
        pad128(activity_bucket), pad128(release_year_bucket),
        pad128(i_avg_rating_bucket), pad128(revenue_bucket),
        Wu1, row(bu1), row(gu_ln), row(bu_ln), Wu2, row(bu2),
        Wi1, row(bi1), row(gi_ln), row(bi_ln), Wi2, row(bi2))
